# Initial kernel scaffold; baseline (speedup 1.0000x reference)
import jax
import jax.numpy as jnp
from jax.experimental import pallas as pl


def kernel(input_, weight):
    raise NotImplementedError("write your pallas kernel here")


# speedup vs baseline: 2.6844x; 2.6844x over previous
import functools

import jax
import jax.numpy as jnp
from jax import lax
from jax.experimental import pallas as pl
from jax.experimental.pallas import tpu as pltpu
from jax.experimental.pallas import tpu_sc as plsc

EMB = 64
SUB = 128
CHUNK = 512
GATHERS = CHUNK // SUB


@functools.partial(jax.jit, static_argnames=())
def _embed_gather(idx2d, weight):
    n_idx_rows, _ = idx2d.shape
    B = n_idx_rows * SUB
    info = plsc.get_sparse_core_info()
    nw = info.num_cores * info.num_subcores
    b_per_w = B // nw
    rows_per_w = n_idx_rows // nw
    n_chunks = b_per_w // CHUNK

    mesh = plsc.VectorSubcoreMesh(core_axis_name="c", subcore_axis_name="s")

    @functools.partial(
        pl.kernel,
        mesh=mesh,
        compiler_params=pltpu.CompilerParams(use_tc_tiling_on_sc=False),
        out_type=jax.ShapeDtypeStruct((B, EMB), jnp.float32),
        scratch_types=[
            pltpu.VMEM((rows_per_w, SUB), jnp.int32),
            pltpu.VMEM((CHUNK, EMB), jnp.float32),
            pltpu.SemaphoreType.DMA,
        ],
    )
    def k(idx_hbm, w_hbm, out_hbm, idx_v, rows_v, sem):
        wid = lax.axis_index("s") * info.num_cores + lax.axis_index("c")
        base = wid * b_per_w
        pltpu.sync_copy(idx_hbm.at[pl.ds(wid * rows_per_w, rows_per_w)], idx_v)

        def body(c, _):
            cps = []
            for j in range(GATHERS):
                cps.append(pltpu.async_copy(
                    w_hbm.at[idx_v.at[c * GATHERS + j]],
                    rows_v.at[pl.ds(j * SUB, SUB)],
                    sem,
                ))
            for cp in cps:
                cp.wait()
            pltpu.sync_copy(rows_v, out_hbm.at[pl.ds(base + c * CHUNK, CHUNK)])
            return _

        lax.fori_loop(0, n_chunks, body, 0, unroll=False)

    return k(idx2d, weight)


def kernel(input_, weight):
    S0, S1 = input_.shape
    idx2d = input_.reshape(S0 * S1 // SUB, SUB)
    out = _embed_gather(idx2d, weight)
    return out.reshape(S0, S1, EMB)


# speedup vs baseline: 2.6941x; 1.0036x over previous
import functools

import jax
import jax.numpy as jnp
from jax import lax
from jax.experimental import pallas as pl
from jax.experimental.pallas import tpu as pltpu
from jax.experimental.pallas import tpu_sc as plsc

EMB = 64
SUB = 128
CHUNK = 256
GATHERS = CHUNK // SUB
NSLOTS = 4


def _embed_gather(idx2d, weight):
    n_idx_rows, _ = idx2d.shape
    B = n_idx_rows * SUB
    info = plsc.get_sparse_core_info()
    nw = info.num_cores * info.num_subcores
    b_per_w = B // nw
    rows_per_w = n_idx_rows // nw
    n_chunks = b_per_w // CHUNK

    mesh = plsc.VectorSubcoreMesh(core_axis_name="c", subcore_axis_name="s")

    @functools.partial(
        pl.kernel,
        mesh=mesh,
        compiler_params=pltpu.CompilerParams(use_tc_tiling_on_sc=False),
        out_type=jax.ShapeDtypeStruct((B, EMB), jnp.float32),
        scratch_types=[
            pltpu.VMEM((rows_per_w, SUB), jnp.int32),
            pltpu.VMEM((NSLOTS, CHUNK, EMB), jnp.float32),
            pltpu.SemaphoreType.DMA((NSLOTS,)),
            pltpu.SemaphoreType.DMA((NSLOTS,)),
        ],
    )
    def k(idx_hbm, w_hbm, out_hbm, idx_v, rows_v, sem_g, sem_w):
        wid = lax.axis_index("s") * info.num_cores + lax.axis_index("c")
        base = wid * b_per_w
        pltpu.sync_copy(idx_hbm.at[pl.ds(wid * rows_per_w, rows_per_w)], idx_v)

        def g_copy(c, s, j):
            return pltpu.make_async_copy(
                w_hbm.at[idx_v.at[c * GATHERS + j]],
                rows_v.at[s, pl.ds(j * SUB, SUB)],
                sem_g.at[s],
            )

        def w_copy(c, s):
            return pltpu.make_async_copy(
                rows_v.at[s],
                out_hbm.at[pl.ds(base + c * CHUNK, CHUNK)],
                sem_w.at[s],
            )

        def fire_g(c, s):
            for j in range(GATHERS):
                g_copy(c, s, j).start()

        for s in range(NSLOTS):
            fire_g(s, s)

        def body(t, _):
            for s in range(NSLOTS):
                c = t * NSLOTS + s
                for j in range(GATHERS):
                    g_copy(c, s, j).wait()
                w_copy(c, s).start()
            for s in range(NSLOTS):
                c = t * NSLOTS + s
                w_copy(c, s).wait()
                fire_g(c + NSLOTS, s)
            return _

        lax.fori_loop(0, n_chunks // NSLOTS - 1, body, 0, unroll=False)

        for s in range(NSLOTS):
            c = n_chunks - NSLOTS + s
            for j in range(GATHERS):
                g_copy(c, s, j).wait()
            w_copy(c, s).start()
        for s in range(NSLOTS):
            c = n_chunks - NSLOTS + s
            w_copy(c, s).wait()

    return k(idx2d, weight)


def kernel(input_, weight):
    S0, S1 = input_.shape
    idx2d = input_.reshape(S0 * S1 // SUB, SUB)
    out = _embed_gather(idx2d, weight)
    return out.reshape(S0, S1, EMB)
